# baseline (speedup 1.0000x reference)
import functools

import jax
import jax.numpy as jnp
from jax import lax
from jax.experimental import pallas as pl
from jax.experimental.pallas import tpu as pltpu
from jax.experimental.pallas import tpu_sc as plsc

B = 320000
V = 10000
D = 128
NC = 2
NS = 16
NW = NC * NS
PER_W = B // NW
CHUNK = 80
NCHUNK = PER_W // CHUNK

SHARE = 624
TAIL = V - NS * SHARE


def _gather_body(idx_hbm, x_hbm, out_hbm, idx_v, shared, buf0, buf1, buf2,
                 gsem0, gsem1, gsem2, ssem0, ssem1, ssem2):
    bufs = (buf0, buf1, buf2)
    gsems = (gsem0, gsem1, gsem2)
    ssems = (ssem0, ssem1, ssem2)
    sid = lax.axis_index("s")
    wid = sid * NC + lax.axis_index("c")
    base = wid * PER_W

    pltpu.sync_copy(idx_hbm.at[pl.ds(base, PER_W)], idx_v)
    srow = pl.multiple_of(sid * SHARE, SHARE)
    pltpu.sync_copy(x_hbm.at[pl.ds(srow, SHARE)], shared.at[pl.ds(srow, SHARE)])

    @pl.when(sid == NS - 1)
    def _():
        pltpu.sync_copy(x_hbm.at[pl.ds(NS * SHARE, TAIL)],
                        shared.at[pl.ds(NS * SHARE, TAIL)])

    plsc.subcore_barrier()

    def fire_gather(m, b):
        off = pl.multiple_of(m * CHUNK, CHUNK)
        pltpu.async_copy(
            shared.at[idx_v.at[pl.ds(off, CHUNK)]], bufs[b], gsems[b])

    def drain_gather(m, b):
        off = pl.multiple_of(m * CHUNK, CHUNK)
        pltpu.make_async_copy(
            shared.at[idx_v.at[pl.ds(off, CHUNK)]], bufs[b], gsems[b]).wait()

    def start_store(m, b):
        off = pl.multiple_of(m * CHUNK, CHUNK)
        pltpu.async_copy(bufs[b], out_hbm.at[pl.ds(base + off, CHUNK)], ssems[b])

    def wait_store(m, b):
        off = pl.multiple_of(m * CHUNK, CHUNK)
        pltpu.make_async_copy(
            bufs[b], out_hbm.at[pl.ds(base + off, CHUNK)], ssems[b]).wait()

    fire_gather(0, 0)
    fire_gather(1, 1)

    def step(m, carry):
        def do(b, n):
            pl.when(m >= 2)(lambda: wait_store(m - 2, n))

            @pl.when(m < NCHUNK - 1)
            def _():
                fire_gather(m + 1, n)

            drain_gather(m, b)
            start_store(m, b)
        pl.when(m % 3 == 0)(lambda: do(0, 1))
        pl.when(m % 3 == 1)(lambda: do(1, 2))
        pl.when(m % 3 == 2)(lambda: do(2, 0))
        return carry

    lax.fori_loop(0, NCHUNK, step, 0, unroll=False)
    wait_store(NCHUNK - 2, (NCHUNK - 2) % 3)
    wait_store(NCHUNK - 1, (NCHUNK - 1) % 3)


@jax.jit
def _gather(indexes, x):
    mesh = plsc.VectorSubcoreMesh(core_axis_name="c", subcore_axis_name="s")
    kfn = functools.partial(
        pl.kernel,
        mesh=mesh,
        out_type=jax.ShapeDtypeStruct((B, D), jnp.float32),
        scratch_types=(
            [pltpu.VMEM((PER_W,), jnp.int32)]
            + [pltpu.VMEM_SHARED((V, D), jnp.float32)]
            + [pltpu.VMEM((CHUNK, D), jnp.float32) for _ in range(3)]
            + [pltpu.SemaphoreType.DMA for _ in range(6)]
        ),
    )(_gather_body)
    return kfn(indexes, x)


def kernel(indexes, x):
    return _gather(indexes, x)


# speedup vs baseline: 1.0709x; 1.0709x over previous
import functools

import jax
import jax.numpy as jnp
from jax import lax
from jax.experimental import pallas as pl
from jax.experimental.pallas import tpu as pltpu
from jax.experimental.pallas import tpu_sc as plsc

B = 320000
V = 10000
D = 128
NC = 2
NS = 16
NW = NC * NS
PER_W = B // NW
CHUNK = 80
NCHUNK = PER_W // CHUNK

SHARE = 624
TAIL = V - NS * SHARE
HBM_WARM = 8


def _gather_body(idx_hbm, x_hbm, out_hbm, idx_v, shared, buf0, buf1,
                 gsem0, gsem1, ssem0, ssem1, psem):
    bufs = (buf0, buf1)
    gsems = (gsem0, gsem1)
    ssems = (ssem0, ssem1)
    sid = lax.axis_index("s")
    wid = sid * NC + lax.axis_index("c")
    base = wid * PER_W

    srow = pl.multiple_of(sid * SHARE, SHARE)
    pltpu.async_copy(x_hbm.at[pl.ds(srow, SHARE)],
                     shared.at[pl.ds(srow, SHARE)], psem)
    @pl.when(sid == NS - 1)
    def _():
        pltpu.async_copy(x_hbm.at[pl.ds(NS * SHARE, TAIL)],
                         shared.at[pl.ds(NS * SHARE, TAIL)], psem)
    pltpu.sync_copy(idx_hbm.at[pl.ds(base, PER_W)], idx_v)

    def wait_preload():
        pltpu.make_async_copy(x_hbm.at[pl.ds(srow, SHARE)],
                              shared.at[pl.ds(srow, SHARE)], psem).wait()
        @pl.when(sid == NS - 1)
        def _():
            pltpu.make_async_copy(x_hbm.at[pl.ds(NS * SHARE, TAIL)],
                                  shared.at[pl.ds(NS * SHARE, TAIL)],
                                  psem).wait()

    def fire_gather(m, b, src):
        off = pl.multiple_of(m * CHUNK, CHUNK)
        pltpu.async_copy(src.at[idx_v.at[pl.ds(off, CHUNK)]], bufs[b], gsems[b])

    def drain_gather(m, b, src):
        off = pl.multiple_of(m * CHUNK, CHUNK)
        pltpu.make_async_copy(
            src.at[idx_v.at[pl.ds(off, CHUNK)]], bufs[b], gsems[b]).wait()

    def start_store(m, b):
        off = pl.multiple_of(m * CHUNK, CHUNK)
        pltpu.async_copy(bufs[b], out_hbm.at[pl.ds(base + off, CHUNK)], ssems[b])

    def wait_store(m, b):
        off = pl.multiple_of(m * CHUNK, CHUNK)
        pltpu.make_async_copy(
            bufs[b], out_hbm.at[pl.ds(base + off, CHUNK)], ssems[b]).wait()

    fire_gather(0, 0, x_hbm)

    for m in range(HBM_WARM):
        b, o = m % 2, 1 - m % 2
        if m >= 1:
            wait_store(m - 1, b=o)
        if m == HBM_WARM - 1:
            wait_preload()
            plsc.subcore_barrier()
        fire_gather(m + 1, o, shared if m + 1 >= HBM_WARM else x_hbm)
        drain_gather(m, b, x_hbm)
        start_store(m, b)

    def step(m, carry):
        def do(b, o):
            def prefetch():
                wait_store(m - 1, o)
                fire_gather(m + 1, o, shared)
            pl.when(m < NCHUNK - 1)(prefetch)
            drain_gather(m, b, shared)
            start_store(m, b)
        pl.when(m % 2 == 0)(lambda: do(0, 1))
        pl.when(m % 2 == 1)(lambda: do(1, 0))
        return carry

    lax.fori_loop(HBM_WARM, NCHUNK, step, 0, unroll=False)
    wait_store(NCHUNK - 2, (NCHUNK - 2) % 2)
    wait_store(NCHUNK - 1, (NCHUNK - 1) % 2)


@jax.jit
def _gather(indexes, x):
    mesh = plsc.VectorSubcoreMesh(core_axis_name="c", subcore_axis_name="s")
    kfn = functools.partial(
        pl.kernel,
        mesh=mesh,
        out_type=jax.ShapeDtypeStruct((B, D), jnp.float32),
        scratch_types=(
            [pltpu.VMEM((PER_W,), jnp.int32)]
            + [pltpu.VMEM_SHARED((V, D), jnp.float32)]
            + [pltpu.VMEM((CHUNK, D), jnp.float32) for _ in range(2)]
            + [pltpu.SemaphoreType.DMA for _ in range(5)]
        ),
    )(_gather_body)
    return kfn(indexes, x)


def kernel(indexes, x):
    return _gather(indexes, x)
